# baseline (speedup 1.0000x reference)
import functools

import jax
import jax.numpy as jnp
from jax import lax
from jax.experimental import pallas as pl
from jax.experimental.pallas import tpu as pltpu
from jax.experimental.pallas import tpu_sc as plsc

NC = 2
NS = 16
NW = NC * NS
L = 16

B = 16384
D = 64
BPW = B // NW
CH = 128
NCH = BPW // CH

_mesh = plsc.VectorSubcoreMesh(core_axis_name="c", subcore_axis_name="s")


@functools.partial(
    pl.kernel,
    out_type=jax.ShapeDtypeStruct((B,), jnp.float32),
    mesh=_mesh,
    compiler_params=pltpu.CompilerParams(
        needs_layout_passes=False, use_tc_tiling_on_sc=False),
    scratch_types=[
        pltpu.VMEM((NCH, CH), jnp.int32),
        pltpu.VMEM((NCH, CH), jnp.int32),
        pltpu.VMEM((BPW, D), jnp.float32),
        pltpu.VMEM((BPW, D), jnp.float32),
        pltpu.VMEM((BPW,), jnp.float32),
        pltpu.VMEM((L * (L + 1),), jnp.float32),
        pltpu.SemaphoreType.DMA,
    ],
)
def _mf_sc(u_hbm, v_hbm, user_hbm, item_hbm, out_hbm,
           uidx, vidx, ue, ve, outv, pbuf, sem):
    wid = lax.axis_index("s") * NC + lax.axis_index("c")
    base = wid * BPW

    for c in range(NCH):
        pltpu.sync_copy(u_hbm.at[pl.ds(base + c * CH, CH)], uidx.at[c])
        pltpu.sync_copy(v_hbm.at[pl.ds(base + c * CH, CH)], vidx.at[c])

    copies = []
    for c in range(NCH):
        copies.append(pltpu.async_copy(
            user_hbm.at[uidx.at[c]], ue.at[pl.ds(c * CH, CH)], sem))
        copies.append(pltpu.async_copy(
            item_hbm.at[vidx.at[c]], ve.at[pl.ds(c * CH, CH)], sem))
    for cp in copies:
        cp.wait()

    lane = lax.iota(jnp.int32, L)

    def group_body(g, carry):
        base_r = g * L
        for i in range(L):
            r = base_r + i
            acc = ue[r, pl.ds(0, L)] * ve[r, pl.ds(0, L)]
            for q in range(1, D // L):
                acc = acc + ue[r, pl.ds(q * L, L)] * ve[r, pl.ds(q * L, L)]
            plsc.store_scatter(pbuf, [lane * (L + 1) + i], acc)
        s = pbuf[pl.ds(0, L)]
        for l in range(1, L):
            s = s + pbuf[pl.ds(l * (L + 1), L)]
        outv[pl.ds(base_r, L)] = s
        return carry

    lax.fori_loop(0, BPW // L, group_body, 0)

    pltpu.sync_copy(outv, out_hbm.at[pl.ds(base, BPW)])


def kernel(u, v, user_emb, item_emb):
    return _mf_sc(u, v, user_emb, item_emb)
